# baseline (speedup 1.0000x reference)
import functools

import jax
import jax.numpy as jnp
from jax import lax
from jax.experimental import pallas as pl
from jax.experimental.pallas import tpu as pltpu
from jax.experimental.pallas import tpu_sc as plsc

N_NODES = 10000
N_PAD = 10112
N_EDGES = 320000
D_FEAT = 128
NC = 2
NS = 16
NW = NC * NS
EDGES_PER_TILE = N_EDGES // NW
CHUNK = 128
N_CHUNKS = -(-EDGES_PER_TILE // CHUNK)
PAD_EDGES = N_CHUNKS * CHUNK - EDGES_PER_TILE
ROWS_PER_TILE = N_PAD // NS


def _sc_aggregate(h, edges, zeros):
    mesh = plsc.VectorSubcoreMesh(core_axis_name="c", subcore_axis_name="s")

    @functools.partial(
        pl.kernel,
        mesh=mesh,
        out_type=jax.ShapeDtypeStruct((NC, N_PAD, D_FEAT), jnp.float32),
        scratch_types=[
            pltpu.VMEM((N_CHUNKS, 2, CHUNK), jnp.int32),
            pltpu.VMEM((CHUNK, D_FEAT), jnp.float32),
            pltpu.VMEM_SHARED((N_PAD, D_FEAT), jnp.float32),
        ],
    )
    def k(h_hbm, e_hbm, z_hbm, out_hbm, idx_v, rows_v, acc_sh):
        c = lax.axis_index("c")
        s = lax.axis_index("s")
        wid = c * NS + s
        row0 = s * ROWS_PER_TILE
        pltpu.sync_copy(z_hbm.at[pl.ds(row0, ROWS_PER_TILE)],
                        acc_sh.at[pl.ds(row0, ROWS_PER_TILE)])
        pltpu.sync_copy(e_hbm.at[wid], idx_v)
        plsc.subcore_barrier()

        @pl.loop(0, N_CHUNKS)
        def _(j):
            pltpu.sync_copy(h_hbm.at[idx_v.at[j, 0]], rows_v)
            pltpu.sync_copy(rows_v, acc_sh.at[idx_v.at[j, 1]], add=True)

        plsc.subcore_barrier()
        pltpu.sync_copy(acc_sh.at[pl.ds(row0, ROWS_PER_TILE)],
                        out_hbm.at[c, pl.ds(row0, ROWS_PER_TILE)])

    return k(h, edges, zeros)


def _sc_degree(ones, edges, zeros):
    mesh = plsc.VectorSubcoreMesh(core_axis_name="c", subcore_axis_name="s")

    @functools.partial(
        pl.kernel,
        mesh=mesh,
        out_type=jax.ShapeDtypeStruct((NC, N_PAD, D_FEAT), jnp.float32),
        scratch_types=[
            pltpu.VMEM((N_CHUNKS, 2, CHUNK), jnp.int32),
            pltpu.VMEM((CHUNK, D_FEAT), jnp.float32),
            pltpu.VMEM_SHARED((N_PAD, D_FEAT), jnp.float32),
        ],
    )
    def k(ones_hbm, e_hbm, z_hbm, out_hbm, idx_v, ones_v, acc_sh):
        c = lax.axis_index("c")
        s = lax.axis_index("s")
        wid = c * NS + s
        row0 = s * ROWS_PER_TILE
        pltpu.sync_copy(z_hbm.at[pl.ds(row0, ROWS_PER_TILE)],
                        acc_sh.at[pl.ds(row0, ROWS_PER_TILE)])
        pltpu.sync_copy(ones_hbm, ones_v)
        pltpu.sync_copy(e_hbm.at[wid], idx_v)
        plsc.subcore_barrier()

        @pl.loop(0, N_CHUNKS)
        def _(j):
            pltpu.sync_copy(ones_v, acc_sh.at[idx_v.at[j, 1]], add=True)

        plsc.subcore_barrier()
        pltpu.sync_copy(acc_sh.at[pl.ds(row0, ROWS_PER_TILE)],
                        out_hbm.at[c, pl.ds(row0, ROWS_PER_TILE)])

    return k(ones, edges, zeros)


def _layer1_body(p_ref, hist_ref, x_ref, wl_ref, wr_ref, b_ref, o_ref, deg_ref):
    deg = jnp.maximum(hist_ref[0, :, :1] + hist_ref[1, :, :1], 1.0)
    agg = (p_ref[0] + p_ref[1]) / deg
    out = (jnp.dot(agg, wl_ref[...], preferred_element_type=jnp.float32)
           + jnp.dot(x_ref[...], wr_ref[...], preferred_element_type=jnp.float32)
           + b_ref[...][None, :])
    o_ref[...] = jnp.maximum(out, 0.0)
    deg_ref[...] = deg


def _layerN_body(act, p_ref, deg_ref, h_ref, wl_ref, wr_ref, b_ref, o_ref):
    agg = (p_ref[0] + p_ref[1]) / deg_ref[...]
    out = (jnp.dot(agg, wl_ref[...], preferred_element_type=jnp.float32)
           + jnp.dot(h_ref[...], wr_ref[...], preferred_element_type=jnp.float32)
           + b_ref[...][None, :])
    if act == "relu":
        out = jnp.maximum(out, 0.0)
    else:
        out = jax.nn.sigmoid(out)
    o_ref[...] = out


_BLK = 1264


def _tc_layer1(p, hist, x, Wl, Wr, b):
    d_out = Wl.shape[1]
    return pl.pallas_call(
        _layer1_body,
        grid=(N_PAD // _BLK,),
        in_specs=[
            pl.BlockSpec((NC, _BLK, D_FEAT), lambda i: (0, i, 0)),
            pl.BlockSpec((NC, _BLK, D_FEAT), lambda i: (0, i, 0)),
            pl.BlockSpec((_BLK, D_FEAT), lambda i: (i, 0)),
            pl.BlockSpec(Wl.shape, lambda i: (0, 0)),
            pl.BlockSpec(Wr.shape, lambda i: (0, 0)),
            pl.BlockSpec(b.shape, lambda i: (0,)),
        ],
        out_specs=[
            pl.BlockSpec((_BLK, d_out), lambda i: (i, 0)),
            pl.BlockSpec((_BLK, 1), lambda i: (i, 0)),
        ],
        out_shape=[
            jax.ShapeDtypeStruct((N_PAD, d_out), jnp.float32),
            jax.ShapeDtypeStruct((N_PAD, 1), jnp.float32),
        ],
    )(p, hist, x, Wl, Wr, b)


def _tc_layerN(p, deg, h, Wl, Wr, b, act):
    d_out = Wl.shape[1]
    return pl.pallas_call(
        functools.partial(_layerN_body, act),
        grid=(N_PAD // _BLK,),
        in_specs=[
            pl.BlockSpec((NC, _BLK, D_FEAT), lambda i: (0, i, 0)),
            pl.BlockSpec((_BLK, 1), lambda i: (i, 0)),
            pl.BlockSpec((_BLK, D_FEAT), lambda i: (i, 0)),
            pl.BlockSpec(Wl.shape, lambda i: (0, 0)),
            pl.BlockSpec(Wr.shape, lambda i: (0, 0)),
            pl.BlockSpec(b.shape, lambda i: (0,)),
        ],
        out_specs=pl.BlockSpec((_BLK, d_out), lambda i: (i, 0)),
        out_shape=jax.ShapeDtypeStruct((N_PAD, d_out), jnp.float32),
    )(p, deg, h, Wl, Wr, b)


def kernel(x, edge_index, Wl1, Wr1, b1, Wl2, Wr2, b2, Wl3, Wr3, b3):
    ei = edge_index.astype(jnp.int32)
    src_t = jnp.pad(ei[0].reshape(NW, EDGES_PER_TILE), ((0, 0), (0, PAD_EDGES)))
    dst_t = jnp.pad(ei[1].reshape(NW, EDGES_PER_TILE), ((0, 0), (0, PAD_EDGES)),
                    constant_values=N_NODES)
    edges = jnp.stack([src_t.reshape(NW, N_CHUNKS, CHUNK),
                       dst_t.reshape(NW, N_CHUNKS, CHUNK)], axis=2)
    xp = jnp.pad(x, ((0, N_PAD - N_NODES), (0, 0)))
    z = jnp.zeros((N_PAD, D_FEAT), jnp.float32)
    ones = jnp.ones((CHUNK, D_FEAT), jnp.float32)

    hist = _sc_degree(ones, edges, z)
    p1 = _sc_aggregate(xp, edges, z)
    h1, deg = _tc_layer1(p1, hist, xp, Wl1, Wr1, b1)
    p2 = _sc_aggregate(h1, edges, z)
    h2 = _tc_layerN(p2, deg, h1, Wl2, Wr2, b2, "relu")
    p3 = _sc_aggregate(h2, edges, z)
    return _tc_layerN(p3, deg, h2, Wl3, Wr3, b3, "sigmoid")[:N_NODES]


# speedup vs baseline: 1.3528x; 1.3528x over previous
import functools

import jax
import jax.numpy as jnp
from jax import lax
from jax.experimental import pallas as pl
from jax.experimental.pallas import tpu as pltpu
from jax.experimental.pallas import tpu_sc as plsc

N_NODES = 10000
N_PAD = 10112
N_EDGES = 320000
D_FEAT = 128
NC = 2
NS = 16
NW = NC * NS
EDGES_PER_TILE = N_EDGES // NW
CHUNK = 80
N_CHUNKS = -(-EDGES_PER_TILE // CHUNK)
PAD_EDGES = N_CHUNKS * CHUNK - EDGES_PER_TILE
ROWS_PER_TILE = N_PAD // NS


def _sc_aggregate(h, edges, zeros):
    mesh = plsc.VectorSubcoreMesh(core_axis_name="c", subcore_axis_name="s")

    @functools.partial(
        pl.kernel,
        mesh=mesh,
        out_type=jax.ShapeDtypeStruct((NC, N_PAD, D_FEAT), jnp.float32),
        scratch_types=[
            pltpu.VMEM((N_CHUNKS, 2, CHUNK), jnp.int32),
            pltpu.VMEM((CHUNK, D_FEAT), jnp.float32),
            pltpu.VMEM_SHARED((N_PAD, D_FEAT), jnp.float32),
        ],
    )
    def k(h_hbm, e_hbm, z_hbm, out_hbm, idx_v, rows_v, acc_sh):
        c = lax.axis_index("c")
        s = lax.axis_index("s")
        wid = c * NS + s
        row0 = s * ROWS_PER_TILE
        pltpu.sync_copy(z_hbm.at[pl.ds(row0, ROWS_PER_TILE)],
                        acc_sh.at[pl.ds(row0, ROWS_PER_TILE)])
        pltpu.sync_copy(e_hbm.at[wid], idx_v)
        plsc.subcore_barrier()

        @pl.loop(0, N_CHUNKS)
        def _(j):
            pltpu.sync_copy(h_hbm.at[idx_v.at[j, 0]], rows_v)
            pltpu.sync_copy(rows_v, acc_sh.at[idx_v.at[j, 1]], add=True)

        plsc.subcore_barrier()
        pltpu.sync_copy(acc_sh.at[pl.ds(row0, ROWS_PER_TILE)],
                        out_hbm.at[c, pl.ds(row0, ROWS_PER_TILE)])

    return k(h, edges, zeros)


def _sc_degree(ones, edges, zeros):
    mesh = plsc.VectorSubcoreMesh(core_axis_name="c", subcore_axis_name="s")

    @functools.partial(
        pl.kernel,
        mesh=mesh,
        out_type=jax.ShapeDtypeStruct((NC, N_PAD, D_FEAT), jnp.float32),
        scratch_types=[
            pltpu.VMEM((N_CHUNKS, 2, CHUNK), jnp.int32),
            pltpu.VMEM((CHUNK, D_FEAT), jnp.float32),
            pltpu.VMEM_SHARED((N_PAD, D_FEAT), jnp.float32),
        ],
    )
    def k(ones_hbm, e_hbm, z_hbm, out_hbm, idx_v, ones_v, acc_sh):
        c = lax.axis_index("c")
        s = lax.axis_index("s")
        wid = c * NS + s
        row0 = s * ROWS_PER_TILE
        pltpu.sync_copy(z_hbm.at[pl.ds(row0, ROWS_PER_TILE)],
                        acc_sh.at[pl.ds(row0, ROWS_PER_TILE)])
        pltpu.sync_copy(ones_hbm, ones_v)
        pltpu.sync_copy(e_hbm.at[wid], idx_v)
        plsc.subcore_barrier()

        @pl.loop(0, N_CHUNKS)
        def _(j):
            pltpu.sync_copy(ones_v, acc_sh.at[idx_v.at[j, 1]], add=True)

        plsc.subcore_barrier()
        pltpu.sync_copy(acc_sh.at[pl.ds(row0, ROWS_PER_TILE)],
                        out_hbm.at[c, pl.ds(row0, ROWS_PER_TILE)])

    return k(ones, edges, zeros)


def _layer1_body(p_ref, hist_ref, x_ref, wl_ref, wr_ref, b_ref, o_ref, deg_ref):
    deg = jnp.maximum(hist_ref[0, :, :1] + hist_ref[1, :, :1], 1.0)
    agg = (p_ref[0] + p_ref[1]) / deg
    out = (jnp.dot(agg, wl_ref[...], preferred_element_type=jnp.float32)
           + jnp.dot(x_ref[...], wr_ref[...], preferred_element_type=jnp.float32)
           + b_ref[...][None, :])
    o_ref[...] = jnp.maximum(out, 0.0)
    deg_ref[...] = deg


def _layerN_body(act, p_ref, deg_ref, h_ref, wl_ref, wr_ref, b_ref, o_ref):
    agg = (p_ref[0] + p_ref[1]) / deg_ref[...]
    out = (jnp.dot(agg, wl_ref[...], preferred_element_type=jnp.float32)
           + jnp.dot(h_ref[...], wr_ref[...], preferred_element_type=jnp.float32)
           + b_ref[...][None, :])
    if act == "relu":
        out = jnp.maximum(out, 0.0)
    else:
        out = jax.nn.sigmoid(out)
    o_ref[...] = out


_BLK = 1264


def _tc_layer1(p, hist, x, Wl, Wr, b):
    d_out = Wl.shape[1]
    return pl.pallas_call(
        _layer1_body,
        grid=(N_PAD // _BLK,),
        in_specs=[
            pl.BlockSpec((NC, _BLK, D_FEAT), lambda i: (0, i, 0)),
            pl.BlockSpec((NC, _BLK, D_FEAT), lambda i: (0, i, 0)),
            pl.BlockSpec((_BLK, D_FEAT), lambda i: (i, 0)),
            pl.BlockSpec(Wl.shape, lambda i: (0, 0)),
            pl.BlockSpec(Wr.shape, lambda i: (0, 0)),
            pl.BlockSpec(b.shape, lambda i: (0,)),
        ],
        out_specs=[
            pl.BlockSpec((_BLK, d_out), lambda i: (i, 0)),
            pl.BlockSpec((_BLK, 1), lambda i: (i, 0)),
        ],
        out_shape=[
            jax.ShapeDtypeStruct((N_PAD, d_out), jnp.float32),
            jax.ShapeDtypeStruct((N_PAD, 1), jnp.float32),
        ],
    )(p, hist, x, Wl, Wr, b)


def _tc_layerN(p, deg, h, Wl, Wr, b, act):
    d_out = Wl.shape[1]
    return pl.pallas_call(
        functools.partial(_layerN_body, act),
        grid=(N_PAD // _BLK,),
        in_specs=[
            pl.BlockSpec((NC, _BLK, D_FEAT), lambda i: (0, i, 0)),
            pl.BlockSpec((_BLK, 1), lambda i: (i, 0)),
            pl.BlockSpec((_BLK, D_FEAT), lambda i: (i, 0)),
            pl.BlockSpec(Wl.shape, lambda i: (0, 0)),
            pl.BlockSpec(Wr.shape, lambda i: (0, 0)),
            pl.BlockSpec(b.shape, lambda i: (0,)),
        ],
        out_specs=pl.BlockSpec((_BLK, d_out), lambda i: (i, 0)),
        out_shape=jax.ShapeDtypeStruct((N_PAD, d_out), jnp.float32),
    )(p, deg, h, Wl, Wr, b)


def kernel(x, edge_index, Wl1, Wr1, b1, Wl2, Wr2, b2, Wl3, Wr3, b3):
    ei = edge_index.astype(jnp.int32)
    src_t = jnp.pad(ei[0].reshape(NW, EDGES_PER_TILE), ((0, 0), (0, PAD_EDGES)))
    dst_t = jnp.pad(ei[1].reshape(NW, EDGES_PER_TILE), ((0, 0), (0, PAD_EDGES)),
                    constant_values=N_NODES)
    edges = jnp.stack([src_t.reshape(NW, N_CHUNKS, CHUNK),
                       dst_t.reshape(NW, N_CHUNKS, CHUNK)], axis=2)
    xp = jnp.pad(x, ((0, N_PAD - N_NODES), (0, 0)))
    z = jnp.zeros((N_PAD, D_FEAT), jnp.float32)
    ones = jnp.ones((CHUNK, D_FEAT), jnp.float32)

    hist = _sc_degree(ones, edges, z)
    p1 = _sc_aggregate(xp, edges, z)
    h1, deg = _tc_layer1(p1, hist, xp, Wl1, Wr1, b1)
    p2 = _sc_aggregate(h1, edges, z)
    h2 = _tc_layerN(p2, deg, h1, Wl2, Wr2, b2, "relu")
    p3 = _sc_aggregate(h2, edges, z)
    return _tc_layerN(p3, deg, h2, Wl3, Wr3, b3, "sigmoid")[:N_NODES]


# speedup vs baseline: 1.4144x; 1.0455x over previous
import functools

import jax
import jax.numpy as jnp
from jax import lax
from jax.experimental import pallas as pl
from jax.experimental.pallas import tpu as pltpu
from jax.experimental.pallas import tpu_sc as plsc

N_NODES = 10000
N_PAD = 10112
N_EDGES = 320000
D_FEAT = 128
NC = 2
NS = 16
NW = NC * NS
EDGES_PER_TILE = N_EDGES // NW
CHUNK = 80
N_CHUNKS = -(-EDGES_PER_TILE // CHUNK)
HALF = CHUNK // 2
PAD_EDGES = N_CHUNKS * CHUNK - EDGES_PER_TILE
ROWS_PER_TILE = N_PAD // NS


def _sc_aggregate(h, edges, zeros):
    mesh = plsc.VectorSubcoreMesh(core_axis_name="c", subcore_axis_name="s")

    @functools.partial(
        pl.kernel,
        mesh=mesh,
        out_type=jax.ShapeDtypeStruct((NC, N_PAD, D_FEAT), jnp.float32),
        scratch_types=[
            pltpu.VMEM((N_CHUNKS, 2, CHUNK), jnp.int32),
            pltpu.VMEM((HALF, D_FEAT), jnp.float32),
            pltpu.VMEM((HALF, D_FEAT), jnp.float32),
            pltpu.VMEM_SHARED((N_PAD, D_FEAT), jnp.float32),
            pltpu.SemaphoreType.DMA,
            pltpu.SemaphoreType.DMA,
        ],
    )
    def k(h_hbm, e_hbm, z_hbm, out_hbm, idx_v, rowsa_v, rowsb_v, acc_sh,
          ga, gb):
        c = lax.axis_index("c")
        s = lax.axis_index("s")
        wid = c * NS + s
        row0 = s * ROWS_PER_TILE
        pltpu.sync_copy(z_hbm.at[pl.ds(row0, ROWS_PER_TILE)],
                        acc_sh.at[pl.ds(row0, ROWS_PER_TILE)])
        pltpu.sync_copy(e_hbm.at[wid], idx_v)
        plsc.subcore_barrier()

        @pl.loop(0, N_CHUNKS)
        def _(j):
            pltpu.async_copy(h_hbm.at[idx_v.at[j, 0, pl.ds(0, HALF)]],
                             rowsa_v, ga)
            pltpu.async_copy(h_hbm.at[idx_v.at[j, 0, pl.ds(HALF, HALF)]],
                             rowsb_v, gb)
            pltpu.make_async_copy(h_hbm.at[idx_v.at[j, 0, pl.ds(0, HALF)]],
                                  rowsa_v, ga).wait()
            pltpu.sync_copy(rowsa_v, acc_sh.at[idx_v.at[j, 1, pl.ds(0, HALF)]],
                            add=True)
            pltpu.make_async_copy(h_hbm.at[idx_v.at[j, 0, pl.ds(HALF, HALF)]],
                                  rowsb_v, gb).wait()
            pltpu.sync_copy(rowsb_v, acc_sh.at[idx_v.at[j, 1, pl.ds(HALF, HALF)]],
                            add=True)

        plsc.subcore_barrier()
        pltpu.sync_copy(acc_sh.at[pl.ds(row0, ROWS_PER_TILE)],
                        out_hbm.at[c, pl.ds(row0, ROWS_PER_TILE)])

    return k(h, edges, zeros)


def _sc_degree(ones, edges, zeros):
    mesh = plsc.VectorSubcoreMesh(core_axis_name="c", subcore_axis_name="s")

    @functools.partial(
        pl.kernel,
        mesh=mesh,
        out_type=jax.ShapeDtypeStruct((NC, N_PAD, D_FEAT), jnp.float32),
        scratch_types=[
            pltpu.VMEM((N_CHUNKS, 2, CHUNK), jnp.int32),
            pltpu.VMEM((CHUNK, D_FEAT), jnp.float32),
            pltpu.VMEM_SHARED((N_PAD, D_FEAT), jnp.float32),
        ],
    )
    def k(ones_hbm, e_hbm, z_hbm, out_hbm, idx_v, ones_v, acc_sh):
        c = lax.axis_index("c")
        s = lax.axis_index("s")
        wid = c * NS + s
        row0 = s * ROWS_PER_TILE
        pltpu.sync_copy(z_hbm.at[pl.ds(row0, ROWS_PER_TILE)],
                        acc_sh.at[pl.ds(row0, ROWS_PER_TILE)])
        pltpu.sync_copy(ones_hbm, ones_v)
        pltpu.sync_copy(e_hbm.at[wid], idx_v)
        plsc.subcore_barrier()

        @pl.loop(0, N_CHUNKS)
        def _(j):
            pltpu.sync_copy(ones_v, acc_sh.at[idx_v.at[j, 1]], add=True)

        plsc.subcore_barrier()
        pltpu.sync_copy(acc_sh.at[pl.ds(row0, ROWS_PER_TILE)],
                        out_hbm.at[c, pl.ds(row0, ROWS_PER_TILE)])

    return k(ones, edges, zeros)


def _layer1_body(p_ref, hist_ref, x_ref, wl_ref, wr_ref, b_ref, o_ref, deg_ref):
    deg = jnp.maximum(hist_ref[0, :, :1] + hist_ref[1, :, :1], 1.0)
    agg = (p_ref[0] + p_ref[1]) / deg
    out = (jnp.dot(agg, wl_ref[...], preferred_element_type=jnp.float32)
           + jnp.dot(x_ref[...], wr_ref[...], preferred_element_type=jnp.float32)
           + b_ref[...][None, :])
    o_ref[...] = jnp.maximum(out, 0.0)
    deg_ref[...] = deg


def _layerN_body(act, p_ref, deg_ref, h_ref, wl_ref, wr_ref, b_ref, o_ref):
    agg = (p_ref[0] + p_ref[1]) / deg_ref[...]
    out = (jnp.dot(agg, wl_ref[...], preferred_element_type=jnp.float32)
           + jnp.dot(h_ref[...], wr_ref[...], preferred_element_type=jnp.float32)
           + b_ref[...][None, :])
    if act == "relu":
        out = jnp.maximum(out, 0.0)
    else:
        out = jax.nn.sigmoid(out)
    o_ref[...] = out


_BLK = 2528


def _tc_layer1(p, hist, x, Wl, Wr, b):
    d_out = Wl.shape[1]
    return pl.pallas_call(
        _layer1_body,
        grid=(N_PAD // _BLK,),
        in_specs=[
            pl.BlockSpec((NC, _BLK, D_FEAT), lambda i: (0, i, 0)),
            pl.BlockSpec((NC, _BLK, D_FEAT), lambda i: (0, i, 0)),
            pl.BlockSpec((_BLK, D_FEAT), lambda i: (i, 0)),
            pl.BlockSpec(Wl.shape, lambda i: (0, 0)),
            pl.BlockSpec(Wr.shape, lambda i: (0, 0)),
            pl.BlockSpec(b.shape, lambda i: (0,)),
        ],
        out_specs=[
            pl.BlockSpec((_BLK, d_out), lambda i: (i, 0)),
            pl.BlockSpec((_BLK, 1), lambda i: (i, 0)),
        ],
        out_shape=[
            jax.ShapeDtypeStruct((N_PAD, d_out), jnp.float32),
            jax.ShapeDtypeStruct((N_PAD, 1), jnp.float32),
        ],
    )(p, hist, x, Wl, Wr, b)


def _tc_layerN(p, deg, h, Wl, Wr, b, act):
    d_out = Wl.shape[1]
    return pl.pallas_call(
        functools.partial(_layerN_body, act),
        grid=(N_PAD // _BLK,),
        in_specs=[
            pl.BlockSpec((NC, _BLK, D_FEAT), lambda i: (0, i, 0)),
            pl.BlockSpec((_BLK, 1), lambda i: (i, 0)),
            pl.BlockSpec((_BLK, D_FEAT), lambda i: (i, 0)),
            pl.BlockSpec(Wl.shape, lambda i: (0, 0)),
            pl.BlockSpec(Wr.shape, lambda i: (0, 0)),
            pl.BlockSpec(b.shape, lambda i: (0,)),
        ],
        out_specs=pl.BlockSpec((_BLK, d_out), lambda i: (i, 0)),
        out_shape=jax.ShapeDtypeStruct((N_PAD, d_out), jnp.float32),
    )(p, deg, h, Wl, Wr, b)


def kernel(x, edge_index, Wl1, Wr1, b1, Wl2, Wr2, b2, Wl3, Wr3, b3):
    ei = edge_index.astype(jnp.int32)
    src_t = jnp.pad(ei[0].reshape(NW, EDGES_PER_TILE), ((0, 0), (0, PAD_EDGES)))
    dst_t = jnp.pad(ei[1].reshape(NW, EDGES_PER_TILE), ((0, 0), (0, PAD_EDGES)),
                    constant_values=N_NODES)
    edges = jnp.stack([src_t.reshape(NW, N_CHUNKS, CHUNK),
                       dst_t.reshape(NW, N_CHUNKS, CHUNK)], axis=2)
    xp = jnp.pad(x, ((0, N_PAD - N_NODES), (0, 0)))
    z = jnp.zeros((N_PAD, D_FEAT), jnp.float32)
    ones = jnp.ones((CHUNK, D_FEAT), jnp.float32)

    hist = _sc_degree(ones, edges, z)
    p1 = _sc_aggregate(xp, edges, z)
    h1, deg = _tc_layer1(p1, hist, xp, Wl1, Wr1, b1)
    p2 = _sc_aggregate(h1, edges, z)
    h2 = _tc_layerN(p2, deg, h1, Wl2, Wr2, b2, "relu")
    p3 = _sc_aggregate(h2, edges, z)
    return _tc_layerN(p3, deg, h2, Wl3, Wr3, b3, "sigmoid")[:N_NODES]


# speedup vs baseline: 1.7298x; 1.2230x over previous
import functools

import jax
import jax.numpy as jnp
from jax import lax
from jax.experimental import pallas as pl
from jax.experimental.pallas import tpu as pltpu
from jax.experimental.pallas import tpu_sc as plsc

N_NODES = 10000
N_PAD = 10112
N_EDGES = 320000
D_FEAT = 128
NC = 2
NS = 16
NW = NC * NS
EDGES_PER_TILE = N_EDGES // NW
CHUNK = 80
N_CHUNKS = -(-EDGES_PER_TILE // CHUNK)
HALF = CHUNK // 2
PAD_EDGES = N_CHUNKS * CHUNK - EDGES_PER_TILE
ROWS_PER_TILE = N_PAD // NS


def _unpack_idx(packed_v, j, i_v):
    for k in range(CHUNK // 16):
        p = packed_v[j, pl.ds(k * 16, 16)]
        i_v[0, pl.ds(k * 16, 16)] = lax.shift_right_logical(p, 14)
        i_v[1, pl.ds(k * 16, 16)] = lax.bitwise_and(p, 16383)


def _sc_aggregate(h, edges, zeros):
    mesh = plsc.VectorSubcoreMesh(core_axis_name="c", subcore_axis_name="s")

    @functools.partial(
        pl.kernel,
        mesh=mesh,
        out_type=jax.ShapeDtypeStruct((NC, N_PAD, D_FEAT), jnp.float32),
        scratch_types=[
            pltpu.VMEM((N_CHUNKS, CHUNK), jnp.int32),
            pltpu.VMEM((2, CHUNK), jnp.int32),
            pltpu.VMEM((2, CHUNK), jnp.int32),
            pltpu.VMEM((CHUNK, D_FEAT), jnp.float32),
            pltpu.VMEM((CHUNK, D_FEAT), jnp.float32),
            pltpu.VMEM_SHARED((N_PAD, D_FEAT), jnp.float32),
            pltpu.SemaphoreType.DMA,
            pltpu.SemaphoreType.DMA,
            pltpu.SemaphoreType.DMA,
            pltpu.SemaphoreType.DMA,
        ],
    )
    def k(h_hbm, e_hbm, z_hbm, out_hbm, packed_v, i0, i1, rows0, rows1,
          acc_sh, g0, g1, s0, s1):
        c = lax.axis_index("c")
        s = lax.axis_index("s")
        wid = c * NS + s
        row0 = s * ROWS_PER_TILE
        pltpu.sync_copy(z_hbm.at[pl.ds(row0, ROWS_PER_TILE)],
                        acc_sh.at[pl.ds(row0, ROWS_PER_TILE)])
        pltpu.sync_copy(e_hbm.at[wid], packed_v)
        plsc.subcore_barrier()

        @pl.loop(0, N_CHUNKS, step=2)
        def _(j):
            @pl.when(j > 0)
            def _():
                pltpu.make_async_copy(rows0, acc_sh.at[i0.at[1]], s0).wait()
            _unpack_idx(packed_v, j, i0)
            pltpu.async_copy(h_hbm.at[i0.at[0]], rows0, g0)

            @pl.when(j + 1 < N_CHUNKS)
            def _():
                @pl.when(j > 0)
                def _():
                    pltpu.make_async_copy(rows1, acc_sh.at[i1.at[1]], s1).wait()
                _unpack_idx(packed_v, j + 1, i1)
                pltpu.async_copy(h_hbm.at[i1.at[0]], rows1, g1)

            pltpu.make_async_copy(h_hbm.at[i0.at[0]], rows0, g0).wait()
            pltpu.async_copy(rows0, acc_sh.at[i0.at[1]], s0, add=True)

            @pl.when(j + 1 < N_CHUNKS)
            def _():
                pltpu.make_async_copy(h_hbm.at[i1.at[0]], rows1, g1).wait()
                pltpu.async_copy(rows1, acc_sh.at[i1.at[1]], s1, add=True)

        pltpu.make_async_copy(rows0, acc_sh.at[i0.at[1]], s0).wait()
        if N_CHUNKS > 1:
            pltpu.make_async_copy(rows1, acc_sh.at[i1.at[1]], s1).wait()
        plsc.subcore_barrier()
        pltpu.sync_copy(acc_sh.at[pl.ds(row0, ROWS_PER_TILE)],
                        out_hbm.at[c, pl.ds(row0, ROWS_PER_TILE)])

    return k(h, edges, zeros)


def _sc_degree(ones, edges, zeros):
    mesh = plsc.VectorSubcoreMesh(core_axis_name="c", subcore_axis_name="s")

    @functools.partial(
        pl.kernel,
        mesh=mesh,
        out_type=jax.ShapeDtypeStruct((NC, N_PAD, D_FEAT), jnp.float32),
        scratch_types=[
            pltpu.VMEM((N_CHUNKS, CHUNK), jnp.int32),
            pltpu.VMEM((2, CHUNK), jnp.int32),
            pltpu.VMEM((CHUNK, D_FEAT), jnp.float32),
            pltpu.VMEM_SHARED((N_PAD, D_FEAT), jnp.float32),
        ],
    )
    def k(ones_hbm, e_hbm, z_hbm, out_hbm, packed_v, i0, ones_v, acc_sh):
        c = lax.axis_index("c")
        s = lax.axis_index("s")
        wid = c * NS + s
        row0 = s * ROWS_PER_TILE
        pltpu.sync_copy(z_hbm.at[pl.ds(row0, ROWS_PER_TILE)],
                        acc_sh.at[pl.ds(row0, ROWS_PER_TILE)])
        pltpu.sync_copy(ones_hbm, ones_v)
        pltpu.sync_copy(e_hbm.at[wid], packed_v)
        plsc.subcore_barrier()

        @pl.loop(0, N_CHUNKS)
        def _(j):
            _unpack_idx(packed_v, j, i0)
            pltpu.sync_copy(ones_v, acc_sh.at[i0.at[1]], add=True)

        plsc.subcore_barrier()
        pltpu.sync_copy(acc_sh.at[pl.ds(row0, ROWS_PER_TILE)],
                        out_hbm.at[c, pl.ds(row0, ROWS_PER_TILE)])

    return k(ones, edges, zeros)


def _layer1_body(p_ref, hist_ref, x_ref, wl_ref, wr_ref, b_ref, o_ref, deg_ref):
    deg = jnp.maximum(hist_ref[0, :, :1] + hist_ref[1, :, :1], 1.0)
    agg = (p_ref[0] + p_ref[1]) / deg
    out = (jnp.dot(agg, wl_ref[...], preferred_element_type=jnp.float32)
           + jnp.dot(x_ref[...], wr_ref[...], preferred_element_type=jnp.float32)
           + b_ref[...][None, :])
    o_ref[...] = jnp.maximum(out, 0.0)
    deg_ref[...] = deg


def _layerN_body(act, p_ref, deg_ref, h_ref, wl_ref, wr_ref, b_ref, o_ref):
    agg = (p_ref[0] + p_ref[1]) / deg_ref[...]
    out = (jnp.dot(agg, wl_ref[...], preferred_element_type=jnp.float32)
           + jnp.dot(h_ref[...], wr_ref[...], preferred_element_type=jnp.float32)
           + b_ref[...][None, :])
    if act == "relu":
        out = jnp.maximum(out, 0.0)
    else:
        out = jax.nn.sigmoid(out)
    o_ref[...] = out


_BLK = 2528


def _tc_layer1(p, hist, x, Wl, Wr, b):
    d_out = Wl.shape[1]
    return pl.pallas_call(
        _layer1_body,
        grid=(N_PAD // _BLK,),
        in_specs=[
            pl.BlockSpec((NC, _BLK, D_FEAT), lambda i: (0, i, 0)),
            pl.BlockSpec((NC, _BLK, D_FEAT), lambda i: (0, i, 0)),
            pl.BlockSpec((_BLK, D_FEAT), lambda i: (i, 0)),
            pl.BlockSpec(Wl.shape, lambda i: (0, 0)),
            pl.BlockSpec(Wr.shape, lambda i: (0, 0)),
            pl.BlockSpec(b.shape, lambda i: (0,)),
        ],
        out_specs=[
            pl.BlockSpec((_BLK, d_out), lambda i: (i, 0)),
            pl.BlockSpec((_BLK, 1), lambda i: (i, 0)),
        ],
        out_shape=[
            jax.ShapeDtypeStruct((N_PAD, d_out), jnp.float32),
            jax.ShapeDtypeStruct((N_PAD, 1), jnp.float32),
        ],
    )(p, hist, x, Wl, Wr, b)


def _tc_layerN(p, deg, h, Wl, Wr, b, act):
    d_out = Wl.shape[1]
    return pl.pallas_call(
        functools.partial(_layerN_body, act),
        grid=(N_PAD // _BLK,),
        in_specs=[
            pl.BlockSpec((NC, _BLK, D_FEAT), lambda i: (0, i, 0)),
            pl.BlockSpec((_BLK, 1), lambda i: (i, 0)),
            pl.BlockSpec((_BLK, D_FEAT), lambda i: (i, 0)),
            pl.BlockSpec(Wl.shape, lambda i: (0, 0)),
            pl.BlockSpec(Wr.shape, lambda i: (0, 0)),
            pl.BlockSpec(b.shape, lambda i: (0,)),
        ],
        out_specs=pl.BlockSpec((_BLK, d_out), lambda i: (i, 0)),
        out_shape=jax.ShapeDtypeStruct((N_PAD, d_out), jnp.float32),
    )(p, deg, h, Wl, Wr, b)


def kernel(x, edge_index, Wl1, Wr1, b1, Wl2, Wr2, b2, Wl3, Wr3, b3):
    ei = edge_index.astype(jnp.int32)
    src_t = jnp.pad(ei[0].reshape(NW, EDGES_PER_TILE), ((0, 0), (0, PAD_EDGES)))
    dst_t = jnp.pad(ei[1].reshape(NW, EDGES_PER_TILE), ((0, 0), (0, PAD_EDGES)),
                    constant_values=N_NODES)
    edges = (src_t * 16384 + dst_t).reshape(NW, N_CHUNKS, CHUNK)
    xp = jnp.pad(x, ((0, N_PAD - N_NODES), (0, 0)))
    z = jnp.zeros((N_PAD, D_FEAT), jnp.float32)
    ones = jnp.ones((CHUNK, D_FEAT), jnp.float32)

    hist = _sc_degree(ones, edges, z)
    p1 = _sc_aggregate(xp, edges, z)
    h1, deg = _tc_layer1(p1, hist, xp, Wl1, Wr1, b1)
    p2 = _sc_aggregate(h1, edges, z)
    h2 = _tc_layerN(p2, deg, h1, Wl2, Wr2, b2, "relu")
    p3 = _sc_aggregate(h2, edges, z)
    return _tc_layerN(p3, deg, h2, Wl3, Wr3, b3, "sigmoid")[:N_NODES]


# speedup vs baseline: 1.9252x; 1.1130x over previous
import dataclasses
import functools

import jax
import jax.numpy as jnp
from jax import lax
from jax.experimental import pallas as pl
from jax.experimental.pallas import tpu as pltpu
from jax.experimental.pallas import tpu_sc as plsc

N_NODES = 10000
N_PAD = 10112
N_EDGES = 320000
D_FEAT = 128
NC = 2
NS = 16
NW = NC * NS
EDGES_PER_TILE = N_EDGES // NW
CHUNK = 80
N_CHUNKS = -(-EDGES_PER_TILE // CHUNK)
HALF = CHUNK // 2
PAD_EDGES = N_CHUNKS * CHUNK - EDGES_PER_TILE
ROWS_PER_TILE = N_PAD // NS


def _unpack_idx(packed_v, j, i_v):
    for k in range(CHUNK // 16):
        p = packed_v[j, pl.ds(k * 16, 16)]
        i_v[0, pl.ds(k * 16, 16)] = lax.shift_right_logical(p, 14)
        i_v[1, pl.ds(k * 16, 16)] = lax.bitwise_and(p, 16383)


def _sc_aggregate(h, edges, zeros):
    mesh = plsc.VectorSubcoreMesh(core_axis_name="c", subcore_axis_name="s")

    @functools.partial(
        pl.kernel,
        mesh=mesh,
        out_type=jax.ShapeDtypeStruct((NC, N_PAD, D_FEAT), jnp.float32),
        scratch_types=[
            pltpu.VMEM((N_CHUNKS, CHUNK), jnp.int32),
            pltpu.VMEM((2, CHUNK), jnp.int32),
            pltpu.VMEM((2, CHUNK), jnp.int32),
            pltpu.VMEM((CHUNK, D_FEAT), jnp.float32),
            pltpu.VMEM((CHUNK, D_FEAT), jnp.float32),
            pltpu.VMEM_SHARED((N_PAD, D_FEAT), jnp.float32),
            pltpu.SemaphoreType.DMA,
            pltpu.SemaphoreType.DMA,
            pltpu.SemaphoreType.DMA,
            pltpu.SemaphoreType.DMA,
        ],
    )
    def k(h_hbm, e_hbm, z_hbm, out_hbm, packed_v, i0, i1, rows0, rows1,
          acc_sh, g0, g1, s0, s1):
        c = lax.axis_index("c")
        s = lax.axis_index("s")
        wid = c * NS + s
        row0 = s * ROWS_PER_TILE
        pltpu.sync_copy(z_hbm.at[pl.ds(row0, ROWS_PER_TILE)],
                        acc_sh.at[pl.ds(row0, ROWS_PER_TILE)])
        pltpu.sync_copy(e_hbm.at[wid], packed_v)
        plsc.subcore_barrier()

        @pl.loop(0, N_CHUNKS, step=2)
        def _(j):
            @pl.when(j > 0)
            def _():
                pltpu.make_async_copy(rows0, acc_sh.at[i0.at[1]], s0).wait()
            _unpack_idx(packed_v, j, i0)
            pltpu.async_copy(h_hbm.at[i0.at[0]], rows0, g0)

            @pl.when(j + 1 < N_CHUNKS)
            def _():
                @pl.when(j > 0)
                def _():
                    pltpu.make_async_copy(rows1, acc_sh.at[i1.at[1]], s1).wait()
                _unpack_idx(packed_v, j + 1, i1)
                pltpu.async_copy(h_hbm.at[i1.at[0]], rows1, g1)

            pltpu.make_async_copy(h_hbm.at[i0.at[0]], rows0, g0).wait()
            pltpu.async_copy(rows0, acc_sh.at[i0.at[1]], s0, add=True)

            @pl.when(j + 1 < N_CHUNKS)
            def _():
                pltpu.make_async_copy(h_hbm.at[i1.at[0]], rows1, g1).wait()
                pltpu.async_copy(rows1, acc_sh.at[i1.at[1]], s1, add=True)

        pltpu.make_async_copy(rows0, acc_sh.at[i0.at[1]], s0).wait()
        if N_CHUNKS > 1:
            pltpu.make_async_copy(rows1, acc_sh.at[i1.at[1]], s1).wait()
        plsc.subcore_barrier()
        pltpu.sync_copy(acc_sh.at[pl.ds(row0, ROWS_PER_TILE)],
                        out_hbm.at[c, pl.ds(row0, ROWS_PER_TILE)])

    return k(h, edges, zeros)


def _sc_degree(edges):
    mesh = plsc.VectorSubcoreMesh(core_axis_name="c", subcore_axis_name="s")
    cp = pltpu.CompilerParams()
    if "needs_layout_passes" in pltpu.CompilerParams.__dataclass_fields__:
        cp = dataclasses.replace(cp, needs_layout_passes=False)

    @functools.partial(
        pl.kernel,
        mesh=mesh,
        compiler_params=cp,
        out_type=jax.ShapeDtypeStruct((NW, N_PAD), jnp.float32),
        scratch_types=[
            pltpu.VMEM((N_CHUNKS * CHUNK,), jnp.int32),
            pltpu.VMEM((N_PAD,), jnp.float32),
        ],
    )
    def k(e_hbm, out_hbm, packed_v, hist_v):
        c = lax.axis_index("c")
        s = lax.axis_index("s")
        wid = c * NS + s
        pltpu.sync_copy(e_hbm.at[wid], packed_v)

        @pl.loop(0, N_PAD, step=16)
        def _(r):
            hist_v[pl.ds(r, 16)] = jnp.zeros((16,), jnp.float32)

        ones16 = jnp.ones((16,), jnp.float32)

        @pl.loop(0, N_CHUNKS * CHUNK, step=16)
        def _(e):
            p = packed_v[pl.ds(e, 16)]
            d = lax.bitwise_and(p, 16383)
            plsc.addupdate_scatter(hist_v, [d], ones16)

        pltpu.sync_copy(hist_v, out_hbm.at[wid])

    return k(edges.reshape(NW, N_CHUNKS * CHUNK))


def _layer1_body(p_ref, hist_ref, x_ref, wl_ref, wr_ref, b_ref, o_ref, deg_ref):
    deg = jnp.maximum(jnp.sum(hist_ref[...], axis=0), 1.0)[:, None]
    agg = (p_ref[0] + p_ref[1]) / deg
    out = (jnp.dot(agg, wl_ref[...], preferred_element_type=jnp.float32)
           + jnp.dot(x_ref[...], wr_ref[...], preferred_element_type=jnp.float32)
           + b_ref[...][None, :])
    o_ref[...] = jnp.maximum(out, 0.0)
    deg_ref[...] = deg


def _layerN_body(act, p_ref, deg_ref, h_ref, wl_ref, wr_ref, b_ref, o_ref):
    agg = (p_ref[0] + p_ref[1]) / deg_ref[...]
    out = (jnp.dot(agg, wl_ref[...], preferred_element_type=jnp.float32)
           + jnp.dot(h_ref[...], wr_ref[...], preferred_element_type=jnp.float32)
           + b_ref[...][None, :])
    if act == "relu":
        out = jnp.maximum(out, 0.0)
    else:
        out = jax.nn.sigmoid(out)
    o_ref[...] = out


_BLK = N_PAD


def _tc_layer1(p, hist, x, Wl, Wr, b):
    d_out = Wl.shape[1]
    return pl.pallas_call(
        _layer1_body,
        grid=(N_PAD // _BLK,),
        in_specs=[
            pl.BlockSpec((NC, _BLK, D_FEAT), lambda i: (0, i, 0)),
            pl.BlockSpec((NW, N_PAD), lambda i: (0, 0)),
            pl.BlockSpec((_BLK, D_FEAT), lambda i: (i, 0)),
            pl.BlockSpec(Wl.shape, lambda i: (0, 0)),
            pl.BlockSpec(Wr.shape, lambda i: (0, 0)),
            pl.BlockSpec(b.shape, lambda i: (0,)),
        ],
        out_specs=[
            pl.BlockSpec((_BLK, d_out), lambda i: (i, 0)),
            pl.BlockSpec((_BLK, 1), lambda i: (i, 0)),
        ],
        out_shape=[
            jax.ShapeDtypeStruct((N_PAD, d_out), jnp.float32),
            jax.ShapeDtypeStruct((N_PAD, 1), jnp.float32),
        ],
    )(p, hist, x, Wl, Wr, b)


def _tc_layerN(p, deg, h, Wl, Wr, b, act):
    d_out = Wl.shape[1]
    return pl.pallas_call(
        functools.partial(_layerN_body, act),
        grid=(N_PAD // _BLK,),
        in_specs=[
            pl.BlockSpec((NC, _BLK, D_FEAT), lambda i: (0, i, 0)),
            pl.BlockSpec((_BLK, 1), lambda i: (i, 0)),
            pl.BlockSpec((_BLK, D_FEAT), lambda i: (i, 0)),
            pl.BlockSpec(Wl.shape, lambda i: (0, 0)),
            pl.BlockSpec(Wr.shape, lambda i: (0, 0)),
            pl.BlockSpec(b.shape, lambda i: (0,)),
        ],
        out_specs=pl.BlockSpec((_BLK, d_out), lambda i: (i, 0)),
        out_shape=jax.ShapeDtypeStruct((N_PAD, d_out), jnp.float32),
    )(p, deg, h, Wl, Wr, b)


def kernel(x, edge_index, Wl1, Wr1, b1, Wl2, Wr2, b2, Wl3, Wr3, b3):
    ei = edge_index.astype(jnp.int32)
    src_t = jnp.pad(ei[0].reshape(NW, EDGES_PER_TILE), ((0, 0), (0, PAD_EDGES)))
    dst_t = jnp.pad(ei[1].reshape(NW, EDGES_PER_TILE), ((0, 0), (0, PAD_EDGES)),
                    constant_values=N_NODES)
    edges = (src_t * 16384 + dst_t).reshape(NW, N_CHUNKS, CHUNK)
    xp = jnp.pad(x, ((0, N_PAD - N_NODES), (0, 0)))
    z = jnp.zeros((N_PAD, D_FEAT), jnp.float32)

    hist = _sc_degree(edges)
    p1 = _sc_aggregate(xp, edges, z)
    h1, deg = _tc_layer1(p1, hist, xp, Wl1, Wr1, b1)
    p2 = _sc_aggregate(h1, edges, z)
    h2 = _tc_layerN(p2, deg, h1, Wl2, Wr2, b2, "relu")
    p3 = _sc_aggregate(h2, edges, z)
    return _tc_layerN(p3, deg, h2, Wl3, Wr3, b3, "sigmoid")[:N_NODES]


# speedup vs baseline: 1.9394x; 1.0074x over previous
import dataclasses
import functools

import jax
import jax.numpy as jnp
from jax import lax
from jax.experimental import pallas as pl
from jax.experimental.pallas import tpu as pltpu
from jax.experimental.pallas import tpu_sc as plsc

N_NODES = 10000
N_PAD = 10112
N_EDGES = 320000
D_FEAT = 128
NC = 2
NS = 16
NW = NC * NS
EDGES_PER_TILE = N_EDGES // NW
CHUNK = 80
N_CHUNKS = -(-EDGES_PER_TILE // CHUNK)
HALF = CHUNK // 2
PAD_EDGES = N_CHUNKS * CHUNK - EDGES_PER_TILE
ROWS_PER_TILE = N_PAD // NS


def _unpack_idx(packed_v, j, i_v):
    for k in range(CHUNK // 16):
        p = packed_v[j, pl.ds(k * 16, 16)]
        i_v[0, pl.ds(k * 16, 16)] = lax.shift_right_logical(p, 14)
        i_v[1, pl.ds(k * 16, 16)] = lax.bitwise_and(p, 16383)


def _sc_aggregate(h, edges, zeros):
    mesh = plsc.VectorSubcoreMesh(core_axis_name="c", subcore_axis_name="s")

    @functools.partial(
        pl.kernel,
        mesh=mesh,
        out_type=jax.ShapeDtypeStruct((NC, N_PAD, D_FEAT), jnp.float32),
        scratch_types=[
            pltpu.VMEM((N_CHUNKS, CHUNK), jnp.int32),
            pltpu.VMEM((2, CHUNK), jnp.int32),
            pltpu.VMEM((2, CHUNK), jnp.int32),
            pltpu.VMEM((CHUNK, D_FEAT), jnp.float32),
            pltpu.VMEM((CHUNK, D_FEAT), jnp.float32),
            pltpu.VMEM_SHARED((N_PAD, D_FEAT), jnp.float32),
            pltpu.SemaphoreType.DMA,
            pltpu.SemaphoreType.DMA,
            pltpu.SemaphoreType.DMA,
            pltpu.SemaphoreType.DMA,
        ],
    )
    def k(h_hbm, e_hbm, z_hbm, out_hbm, packed_v, i0, i1, rows0, rows1,
          acc_sh, g0, g1, s0, s1):
        c = lax.axis_index("c")
        s = lax.axis_index("s")
        wid = c * NS + s
        row0 = s * ROWS_PER_TILE
        pltpu.sync_copy(z_hbm.at[pl.ds(row0, ROWS_PER_TILE)],
                        acc_sh.at[pl.ds(row0, ROWS_PER_TILE)])
        pltpu.sync_copy(e_hbm.at[wid], packed_v)
        plsc.subcore_barrier()

        @pl.loop(0, N_CHUNKS, step=2)
        def _(j):
            @pl.when(j > 0)
            def _():
                pltpu.make_async_copy(rows0, acc_sh.at[i0.at[1]], s0).wait()
            _unpack_idx(packed_v, j, i0)
            pltpu.async_copy(h_hbm.at[i0.at[0]], rows0, g0)

            @pl.when(j + 1 < N_CHUNKS)
            def _():
                @pl.when(j > 0)
                def _():
                    pltpu.make_async_copy(rows1, acc_sh.at[i1.at[1]], s1).wait()
                _unpack_idx(packed_v, j + 1, i1)
                pltpu.async_copy(h_hbm.at[i1.at[0]], rows1, g1)

            pltpu.make_async_copy(h_hbm.at[i0.at[0]], rows0, g0).wait()
            pltpu.async_copy(rows0, acc_sh.at[i0.at[1]], s0, add=True)

            @pl.when(j + 1 < N_CHUNKS)
            def _():
                pltpu.make_async_copy(h_hbm.at[i1.at[0]], rows1, g1).wait()
                pltpu.async_copy(rows1, acc_sh.at[i1.at[1]], s1, add=True)

        pltpu.make_async_copy(rows0, acc_sh.at[i0.at[1]], s0).wait()
        if N_CHUNKS > 1:
            pltpu.make_async_copy(rows1, acc_sh.at[i1.at[1]], s1).wait()
        plsc.subcore_barrier()
        pltpu.sync_copy(acc_sh.at[pl.ds(row0, ROWS_PER_TILE)],
                        out_hbm.at[c, pl.ds(row0, ROWS_PER_TILE)])

    return k(h, edges, zeros)


def _sc_degree(edges):
    mesh = plsc.VectorSubcoreMesh(core_axis_name="c", subcore_axis_name="s")
    cp = pltpu.CompilerParams()
    if "needs_layout_passes" in pltpu.CompilerParams.__dataclass_fields__:
        cp = dataclasses.replace(cp, needs_layout_passes=False)

    @functools.partial(
        pl.kernel,
        mesh=mesh,
        compiler_params=cp,
        out_type=jax.ShapeDtypeStruct((NW, N_PAD), jnp.float32),
        scratch_types=[
            pltpu.VMEM((N_CHUNKS * CHUNK,), jnp.int32),
            pltpu.VMEM((N_PAD,), jnp.float32),
        ],
    )
    def k(e_hbm, out_hbm, packed_v, hist_v):
        c = lax.axis_index("c")
        s = lax.axis_index("s")
        wid = c * NS + s
        pltpu.sync_copy(e_hbm.at[wid], packed_v)

        @pl.loop(0, N_PAD, step=16)
        def _(r):
            hist_v[pl.ds(r, 16)] = jnp.zeros((16,), jnp.float32)

        ones16 = jnp.ones((16,), jnp.float32)

        @pl.loop(0, N_CHUNKS * CHUNK, step=16)
        def _(e):
            p = packed_v[pl.ds(e, 16)]
            d = lax.bitwise_and(p, 16383)
            plsc.addupdate_scatter(hist_v, [d], ones16)

        pltpu.sync_copy(hist_v, out_hbm.at[wid])

    return k(edges.reshape(NW, N_CHUNKS * CHUNK))


def _layer1_body(p_ref, hist_ref, x_ref, wl_ref, wr_ref, b_ref, o_ref, deg_ref):
    deg = jnp.maximum(jnp.sum(hist_ref[...], axis=1), 1.0)[:, None]
    agg = (p_ref[0] + p_ref[1]) / deg
    out = (jnp.dot(agg, wl_ref[...], preferred_element_type=jnp.float32)
           + jnp.dot(x_ref[...], wr_ref[...], preferred_element_type=jnp.float32)
           + b_ref[...][None, :])
    o_ref[...] = jnp.maximum(out, 0.0)
    deg_ref[...] = deg


def _layerN_body(act, p_ref, deg_ref, h_ref, wl_ref, wr_ref, b_ref, o_ref):
    agg = (p_ref[0] + p_ref[1]) / deg_ref[...]
    out = (jnp.dot(agg, wl_ref[...], preferred_element_type=jnp.float32)
           + jnp.dot(h_ref[...], wr_ref[...], preferred_element_type=jnp.float32)
           + b_ref[...][None, :])
    if act == "relu":
        out = jnp.maximum(out, 0.0)
    else:
        out = jax.nn.sigmoid(out)
    o_ref[...] = out


_BLK = 1264


def _tc_layer1(p, hist, x, Wl, Wr, b):
    d_out = Wl.shape[1]
    return pl.pallas_call(
        _layer1_body,
        grid=(N_PAD // _BLK,),
        in_specs=[
            pl.BlockSpec((NC, _BLK, D_FEAT), lambda i: (0, i, 0)),
            pl.BlockSpec((_BLK, NW), lambda i: (i, 0)),
            pl.BlockSpec((_BLK, D_FEAT), lambda i: (i, 0)),
            pl.BlockSpec(Wl.shape, lambda i: (0, 0)),
            pl.BlockSpec(Wr.shape, lambda i: (0, 0)),
            pl.BlockSpec(b.shape, lambda i: (0,)),
        ],
        out_specs=[
            pl.BlockSpec((_BLK, d_out), lambda i: (i, 0)),
            pl.BlockSpec((_BLK, 1), lambda i: (i, 0)),
        ],
        out_shape=[
            jax.ShapeDtypeStruct((N_PAD, d_out), jnp.float32),
            jax.ShapeDtypeStruct((N_PAD, 1), jnp.float32),
        ],
    )(p, hist, x, Wl, Wr, b)


def _tc_layerN(p, deg, h, Wl, Wr, b, act):
    d_out = Wl.shape[1]
    return pl.pallas_call(
        functools.partial(_layerN_body, act),
        grid=(N_PAD // _BLK,),
        in_specs=[
            pl.BlockSpec((NC, _BLK, D_FEAT), lambda i: (0, i, 0)),
            pl.BlockSpec((_BLK, 1), lambda i: (i, 0)),
            pl.BlockSpec((_BLK, D_FEAT), lambda i: (i, 0)),
            pl.BlockSpec(Wl.shape, lambda i: (0, 0)),
            pl.BlockSpec(Wr.shape, lambda i: (0, 0)),
            pl.BlockSpec(b.shape, lambda i: (0,)),
        ],
        out_specs=pl.BlockSpec((_BLK, d_out), lambda i: (i, 0)),
        out_shape=jax.ShapeDtypeStruct((N_PAD, d_out), jnp.float32),
    )(p, deg, h, Wl, Wr, b)


def kernel(x, edge_index, Wl1, Wr1, b1, Wl2, Wr2, b2, Wl3, Wr3, b3):
    ei = edge_index.astype(jnp.int32)
    src_t = jnp.pad(ei[0].reshape(NW, EDGES_PER_TILE), ((0, 0), (0, PAD_EDGES)))
    dst_t = jnp.pad(ei[1].reshape(NW, EDGES_PER_TILE), ((0, 0), (0, PAD_EDGES)),
                    constant_values=N_NODES)
    edges = (src_t * 16384 + dst_t).reshape(NW, N_CHUNKS, CHUNK)
    xp = jnp.pad(x, ((0, N_PAD - N_NODES), (0, 0)))
    z = jnp.zeros((N_PAD, D_FEAT), jnp.float32)

    hist = _sc_degree(edges)
    p1 = _sc_aggregate(xp, edges, z)
    h1, deg = _tc_layer1(p1, hist.T, xp, Wl1, Wr1, b1)
    p2 = _sc_aggregate(h1, edges, z)
    h2 = _tc_layerN(p2, deg, h1, Wl2, Wr2, b2, "relu")
    p3 = _sc_aggregate(h2, edges, z)
    return _tc_layerN(p3, deg, h2, Wl3, Wr3, b3, "sigmoid")[:N_NODES]


# speedup vs baseline: 1.9650x; 1.0132x over previous
import dataclasses
import functools

import jax
import jax.numpy as jnp
from jax import lax
from jax.experimental import pallas as pl
from jax.experimental.pallas import tpu as pltpu
from jax.experimental.pallas import tpu_sc as plsc

N_NODES = 10000
N_PAD = 10112
N_EDGES = 320000
D_FEAT = 128
NC = 2
NS = 16
NW = NC * NS
EDGES_PER_TILE = N_EDGES // NW
CHUNK = 80
N_CHUNKS = -(-EDGES_PER_TILE // CHUNK)
HALF = CHUNK // 2
PAD_EDGES = N_CHUNKS * CHUNK - EDGES_PER_TILE
ROWS_PER_TILE = N_PAD // NS


def _unpack_idx(packed_v, j, i_v):
    for k in range(CHUNK // 16):
        p = packed_v[j, pl.ds(k * 16, 16)]
        i_v[0, pl.ds(k * 16, 16)] = lax.shift_right_logical(p, 14)
        i_v[1, pl.ds(k * 16, 16)] = lax.bitwise_and(p, 16383)


def _sc_aggregate(h, edges, zeros):
    mesh = plsc.VectorSubcoreMesh(core_axis_name="c", subcore_axis_name="s")

    @functools.partial(
        pl.kernel,
        mesh=mesh,
        out_type=jax.ShapeDtypeStruct((NC, N_PAD, D_FEAT), jnp.float32),
        scratch_types=[
            pltpu.VMEM((N_CHUNKS, CHUNK), jnp.int32),
            pltpu.VMEM((2, CHUNK), jnp.int32),
            pltpu.VMEM((2, CHUNK), jnp.int32),
            pltpu.VMEM((CHUNK, D_FEAT), jnp.float32),
            pltpu.VMEM((CHUNK, D_FEAT), jnp.float32),
            pltpu.VMEM_SHARED((N_PAD, D_FEAT), jnp.float32),
            pltpu.SemaphoreType.DMA,
            pltpu.SemaphoreType.DMA,
            pltpu.SemaphoreType.DMA,
            pltpu.SemaphoreType.DMA,
            pltpu.SemaphoreType.DMA,
        ],
    )
    def k(h_hbm, e_hbm, z_hbm, out_hbm, packed_v, i0, i1, rows0, rows1,
          acc_sh, g0, g1, s0, s1, zs):
        c = lax.axis_index("c")
        s = lax.axis_index("s")
        wid = c * NS + s
        row0 = s * ROWS_PER_TILE
        pltpu.async_copy(z_hbm.at[pl.ds(row0, ROWS_PER_TILE)],
                         acc_sh.at[pl.ds(row0, ROWS_PER_TILE)], zs)
        pltpu.sync_copy(e_hbm.at[wid], packed_v)
        _unpack_idx(packed_v, 0, i0)
        pltpu.async_copy(h_hbm.at[i0.at[0]], rows0, g0)
        _unpack_idx(packed_v, 1, i1)
        pltpu.async_copy(h_hbm.at[i1.at[0]], rows1, g1)
        pltpu.make_async_copy(z_hbm.at[pl.ds(row0, ROWS_PER_TILE)],
                              acc_sh.at[pl.ds(row0, ROWS_PER_TILE)], zs).wait()
        plsc.subcore_barrier()

        @pl.loop(0, N_CHUNKS, step=2)
        def _(j):
            @pl.when(j > 0)
            def _():
                pltpu.make_async_copy(rows0, acc_sh.at[i0.at[1]], s0).wait()
                _unpack_idx(packed_v, j, i0)
                pltpu.async_copy(h_hbm.at[i0.at[0]], rows0, g0)

            @pl.when(jnp.logical_and(j > 0, j + 1 < N_CHUNKS))
            def _():
                pltpu.make_async_copy(rows1, acc_sh.at[i1.at[1]], s1).wait()
                _unpack_idx(packed_v, j + 1, i1)
                pltpu.async_copy(h_hbm.at[i1.at[0]], rows1, g1)

            pltpu.make_async_copy(h_hbm.at[i0.at[0]], rows0, g0).wait()
            pltpu.async_copy(rows0, acc_sh.at[i0.at[1]], s0, add=True)

            @pl.when(j + 1 < N_CHUNKS)
            def _():
                pltpu.make_async_copy(h_hbm.at[i1.at[0]], rows1, g1).wait()
                pltpu.async_copy(rows1, acc_sh.at[i1.at[1]], s1, add=True)

        pltpu.make_async_copy(rows0, acc_sh.at[i0.at[1]], s0).wait()
        if N_CHUNKS > 1:
            pltpu.make_async_copy(rows1, acc_sh.at[i1.at[1]], s1).wait()
        plsc.subcore_barrier()
        pltpu.sync_copy(acc_sh.at[pl.ds(row0, ROWS_PER_TILE)],
                        out_hbm.at[c, pl.ds(row0, ROWS_PER_TILE)])

    return k(h, edges, zeros)


def _sc_degree(edges):
    mesh = plsc.VectorSubcoreMesh(core_axis_name="c", subcore_axis_name="s")
    cp = pltpu.CompilerParams()
    if "needs_layout_passes" in pltpu.CompilerParams.__dataclass_fields__:
        cp = dataclasses.replace(cp, needs_layout_passes=False)

    @functools.partial(
        pl.kernel,
        mesh=mesh,
        compiler_params=cp,
        out_type=jax.ShapeDtypeStruct((NW, N_PAD), jnp.float32),
        scratch_types=[
            pltpu.VMEM((N_CHUNKS * CHUNK,), jnp.int32),
            pltpu.VMEM((N_PAD,), jnp.float32),
        ],
    )
    def k(e_hbm, out_hbm, packed_v, hist_v):
        c = lax.axis_index("c")
        s = lax.axis_index("s")
        wid = c * NS + s
        pltpu.sync_copy(e_hbm.at[wid], packed_v)

        @pl.loop(0, N_PAD, step=16)
        def _(r):
            hist_v[pl.ds(r, 16)] = jnp.zeros((16,), jnp.float32)

        ones16 = jnp.ones((16,), jnp.float32)

        @pl.loop(0, N_CHUNKS * CHUNK, step=16)
        def _(e):
            p = packed_v[pl.ds(e, 16)]
            d = lax.bitwise_and(p, 16383)
            plsc.addupdate_scatter(hist_v, [d], ones16)

        pltpu.sync_copy(hist_v, out_hbm.at[wid])

    return k(edges.reshape(NW, N_CHUNKS * CHUNK))


def _layer1_body(p_ref, hist_ref, x_ref, wl_ref, wr_ref, b_ref, o_ref, deg_ref):
    deg = jnp.maximum(jnp.sum(hist_ref[...], axis=1), 1.0)[:, None]
    agg = (p_ref[0] + p_ref[1]) / deg
    out = (jnp.dot(agg, wl_ref[...], preferred_element_type=jnp.float32)
           + jnp.dot(x_ref[...], wr_ref[...], preferred_element_type=jnp.float32)
           + b_ref[...][None, :])
    o_ref[...] = jnp.maximum(out, 0.0)
    deg_ref[...] = deg


def _layerN_body(act, p_ref, deg_ref, h_ref, wl_ref, wr_ref, b_ref, o_ref):
    agg = (p_ref[0] + p_ref[1]) / deg_ref[...]
    out = (jnp.dot(agg, wl_ref[...], preferred_element_type=jnp.float32)
           + jnp.dot(h_ref[...], wr_ref[...], preferred_element_type=jnp.float32)
           + b_ref[...][None, :])
    if act == "relu":
        out = jnp.maximum(out, 0.0)
    else:
        out = jax.nn.sigmoid(out)
    o_ref[...] = out


_BLK = 1264


def _tc_layer1(p, hist, x, Wl, Wr, b):
    d_out = Wl.shape[1]
    return pl.pallas_call(
        _layer1_body,
        grid=(N_PAD // _BLK,),
        in_specs=[
            pl.BlockSpec((NC, _BLK, D_FEAT), lambda i: (0, i, 0)),
            pl.BlockSpec((_BLK, NW), lambda i: (i, 0)),
            pl.BlockSpec((_BLK, D_FEAT), lambda i: (i, 0)),
            pl.BlockSpec(Wl.shape, lambda i: (0, 0)),
            pl.BlockSpec(Wr.shape, lambda i: (0, 0)),
            pl.BlockSpec(b.shape, lambda i: (0,)),
        ],
        out_specs=[
            pl.BlockSpec((_BLK, d_out), lambda i: (i, 0)),
            pl.BlockSpec((_BLK, 1), lambda i: (i, 0)),
        ],
        out_shape=[
            jax.ShapeDtypeStruct((N_PAD, d_out), jnp.float32),
            jax.ShapeDtypeStruct((N_PAD, 1), jnp.float32),
        ],
    )(p, hist, x, Wl, Wr, b)


def _tc_layerN(p, deg, h, Wl, Wr, b, act):
    d_out = Wl.shape[1]
    return pl.pallas_call(
        functools.partial(_layerN_body, act),
        grid=(N_PAD // _BLK,),
        in_specs=[
            pl.BlockSpec((NC, _BLK, D_FEAT), lambda i: (0, i, 0)),
            pl.BlockSpec((_BLK, 1), lambda i: (i, 0)),
            pl.BlockSpec((_BLK, D_FEAT), lambda i: (i, 0)),
            pl.BlockSpec(Wl.shape, lambda i: (0, 0)),
            pl.BlockSpec(Wr.shape, lambda i: (0, 0)),
            pl.BlockSpec(b.shape, lambda i: (0,)),
        ],
        out_specs=pl.BlockSpec((_BLK, d_out), lambda i: (i, 0)),
        out_shape=jax.ShapeDtypeStruct((N_PAD, d_out), jnp.float32),
    )(p, deg, h, Wl, Wr, b)


def kernel(x, edge_index, Wl1, Wr1, b1, Wl2, Wr2, b2, Wl3, Wr3, b3):
    ei = edge_index.astype(jnp.int32)
    src_t = jnp.pad(ei[0].reshape(NW, EDGES_PER_TILE), ((0, 0), (0, PAD_EDGES)))
    dst_t = jnp.pad(ei[1].reshape(NW, EDGES_PER_TILE), ((0, 0), (0, PAD_EDGES)),
                    constant_values=N_NODES)
    edges = (src_t * 16384 + dst_t).reshape(NW, N_CHUNKS, CHUNK)
    xp = jnp.pad(x, ((0, N_PAD - N_NODES), (0, 0)))
    z = jnp.zeros((N_PAD, D_FEAT), jnp.float32)

    hist = _sc_degree(edges)
    p1 = _sc_aggregate(xp, edges, z)
    h1, deg = _tc_layer1(p1, hist.T, xp, Wl1, Wr1, b1)
    p2 = _sc_aggregate(h1, edges, z)
    h2 = _tc_layerN(p2, deg, h1, Wl2, Wr2, b2, "relu")
    p3 = _sc_aggregate(h2, edges, z)
    return _tc_layerN(p3, deg, h2, Wl3, Wr3, b3, "sigmoid")[:N_NODES]


# speedup vs baseline: 1.9672x; 1.0011x over previous
import dataclasses
import functools

import jax
import jax.numpy as jnp
from jax import lax
from jax.experimental import pallas as pl
from jax.experimental.pallas import tpu as pltpu
from jax.experimental.pallas import tpu_sc as plsc

N_NODES = 10000
N_PAD = 10112
N_EDGES = 320000
D_FEAT = 128
NC = 2
NS = 16
NW = NC * NS
EDGES_PER_TILE = N_EDGES // NW
CHUNK = 80
N_CHUNKS = -(-EDGES_PER_TILE // CHUNK)
HALF = CHUNK // 2


def _gather2(h_hbm, i_v, rows_v, sem):
    pltpu.async_copy(h_hbm.at[i_v.at[0, pl.ds(0, HALF)]],
                     rows_v.at[pl.ds(0, HALF)], sem)
    pltpu.async_copy(h_hbm.at[i_v.at[0, pl.ds(HALF, HALF)]],
                     rows_v.at[pl.ds(HALF, HALF)], sem)


def _gather2_wait(h_hbm, i_v, rows_v, sem):
    pltpu.make_async_copy(h_hbm.at[i_v.at[0, pl.ds(0, HALF)]],
                          rows_v.at[pl.ds(0, HALF)], sem).wait()
    pltpu.make_async_copy(h_hbm.at[i_v.at[0, pl.ds(HALF, HALF)]],
                          rows_v.at[pl.ds(HALF, HALF)], sem).wait()
PAD_EDGES = N_CHUNKS * CHUNK - EDGES_PER_TILE
ROWS_PER_TILE = N_PAD // NS


def _unpack_idx(packed_v, j, i_v):
    for k in range(CHUNK // 16):
        p = packed_v[j, pl.ds(k * 16, 16)]
        i_v[0, pl.ds(k * 16, 16)] = lax.shift_right_logical(p, 14)
        i_v[1, pl.ds(k * 16, 16)] = lax.bitwise_and(p, 16383)


def _sc_aggregate(h, edges, zeros):
    mesh = plsc.VectorSubcoreMesh(core_axis_name="c", subcore_axis_name="s")

    @functools.partial(
        pl.kernel,
        mesh=mesh,
        out_type=jax.ShapeDtypeStruct((NC, N_PAD, D_FEAT), jnp.float32),
        scratch_types=[
            pltpu.VMEM((N_CHUNKS, CHUNK), jnp.int32),
            pltpu.VMEM((2, CHUNK), jnp.int32),
            pltpu.VMEM((2, CHUNK), jnp.int32),
            pltpu.VMEM((CHUNK, D_FEAT), jnp.float32),
            pltpu.VMEM((CHUNK, D_FEAT), jnp.float32),
            pltpu.VMEM_SHARED((N_PAD, D_FEAT), jnp.float32),
            pltpu.SemaphoreType.DMA,
            pltpu.SemaphoreType.DMA,
            pltpu.SemaphoreType.DMA,
            pltpu.SemaphoreType.DMA,
            pltpu.SemaphoreType.DMA,
        ],
    )
    def k(h_hbm, e_hbm, z_hbm, out_hbm, packed_v, i0, i1, rows0, rows1,
          acc_sh, g0, g1, s0, s1, zs):
        c = lax.axis_index("c")
        s = lax.axis_index("s")
        wid = c * NS + s
        row0 = s * ROWS_PER_TILE
        pltpu.async_copy(z_hbm.at[pl.ds(row0, ROWS_PER_TILE)],
                         acc_sh.at[pl.ds(row0, ROWS_PER_TILE)], zs)
        pltpu.sync_copy(e_hbm.at[wid], packed_v)
        _unpack_idx(packed_v, 0, i0)
        _gather2(h_hbm, i0, rows0, g0)
        _unpack_idx(packed_v, 1, i1)
        _gather2(h_hbm, i1, rows1, g1)
        pltpu.make_async_copy(z_hbm.at[pl.ds(row0, ROWS_PER_TILE)],
                              acc_sh.at[pl.ds(row0, ROWS_PER_TILE)], zs).wait()
        plsc.subcore_barrier()

        @pl.loop(0, N_CHUNKS, step=2)
        def _(j):
            @pl.when(j > 0)
            def _():
                pltpu.make_async_copy(rows0, acc_sh.at[i0.at[1]], s0).wait()
                _unpack_idx(packed_v, j, i0)
                _gather2(h_hbm, i0, rows0, g0)

            @pl.when(jnp.logical_and(j > 0, j + 1 < N_CHUNKS))
            def _():
                pltpu.make_async_copy(rows1, acc_sh.at[i1.at[1]], s1).wait()
                _unpack_idx(packed_v, j + 1, i1)
                _gather2(h_hbm, i1, rows1, g1)

            _gather2_wait(h_hbm, i0, rows0, g0)
            pltpu.async_copy(rows0, acc_sh.at[i0.at[1]], s0, add=True)

            @pl.when(j + 1 < N_CHUNKS)
            def _():
                _gather2_wait(h_hbm, i1, rows1, g1)
                pltpu.async_copy(rows1, acc_sh.at[i1.at[1]], s1, add=True)

        pltpu.make_async_copy(rows0, acc_sh.at[i0.at[1]], s0).wait()
        if N_CHUNKS > 1:
            pltpu.make_async_copy(rows1, acc_sh.at[i1.at[1]], s1).wait()
        plsc.subcore_barrier()
        pltpu.sync_copy(acc_sh.at[pl.ds(row0, ROWS_PER_TILE)],
                        out_hbm.at[c, pl.ds(row0, ROWS_PER_TILE)])

    return k(h, edges, zeros)


def _sc_degree(edges):
    mesh = plsc.VectorSubcoreMesh(core_axis_name="c", subcore_axis_name="s")
    cp = pltpu.CompilerParams()
    if "needs_layout_passes" in pltpu.CompilerParams.__dataclass_fields__:
        cp = dataclasses.replace(cp, needs_layout_passes=False)

    @functools.partial(
        pl.kernel,
        mesh=mesh,
        compiler_params=cp,
        out_type=jax.ShapeDtypeStruct((NW, N_PAD), jnp.float32),
        scratch_types=[
            pltpu.VMEM((N_CHUNKS * CHUNK,), jnp.int32),
            pltpu.VMEM((N_PAD,), jnp.float32),
        ],
    )
    def k(e_hbm, out_hbm, packed_v, hist_v):
        c = lax.axis_index("c")
        s = lax.axis_index("s")
        wid = c * NS + s
        pltpu.sync_copy(e_hbm.at[wid], packed_v)

        @pl.loop(0, N_PAD, step=16)
        def _(r):
            hist_v[pl.ds(r, 16)] = jnp.zeros((16,), jnp.float32)

        ones16 = jnp.ones((16,), jnp.float32)

        @pl.loop(0, N_CHUNKS * CHUNK, step=16)
        def _(e):
            p = packed_v[pl.ds(e, 16)]
            d = lax.bitwise_and(p, 16383)
            plsc.addupdate_scatter(hist_v, [d], ones16)

        pltpu.sync_copy(hist_v, out_hbm.at[wid])

    return k(edges.reshape(NW, N_CHUNKS * CHUNK))


def _layer1_body(p_ref, hist_ref, x_ref, wl_ref, wr_ref, b_ref, o_ref, deg_ref):
    deg = jnp.maximum(jnp.sum(hist_ref[...], axis=1), 1.0)[:, None]
    agg = (p_ref[0] + p_ref[1]) / deg
    out = (jnp.dot(agg, wl_ref[...], preferred_element_type=jnp.float32)
           + jnp.dot(x_ref[...], wr_ref[...], preferred_element_type=jnp.float32)
           + b_ref[...][None, :])
    o_ref[...] = jnp.maximum(out, 0.0)
    deg_ref[...] = deg


def _layerN_body(act, p_ref, deg_ref, h_ref, wl_ref, wr_ref, b_ref, o_ref):
    agg = (p_ref[0] + p_ref[1]) / deg_ref[...]
    out = (jnp.dot(agg, wl_ref[...], preferred_element_type=jnp.float32)
           + jnp.dot(h_ref[...], wr_ref[...], preferred_element_type=jnp.float32)
           + b_ref[...][None, :])
    if act == "relu":
        out = jnp.maximum(out, 0.0)
    else:
        out = jax.nn.sigmoid(out)
    o_ref[...] = out


_BLK = 1264


def _tc_layer1(p, hist, x, Wl, Wr, b):
    d_out = Wl.shape[1]
    return pl.pallas_call(
        _layer1_body,
        grid=(N_PAD // _BLK,),
        in_specs=[
            pl.BlockSpec((NC, _BLK, D_FEAT), lambda i: (0, i, 0)),
            pl.BlockSpec((_BLK, NW), lambda i: (i, 0)),
            pl.BlockSpec((_BLK, D_FEAT), lambda i: (i, 0)),
            pl.BlockSpec(Wl.shape, lambda i: (0, 0)),
            pl.BlockSpec(Wr.shape, lambda i: (0, 0)),
            pl.BlockSpec(b.shape, lambda i: (0,)),
        ],
        out_specs=[
            pl.BlockSpec((_BLK, d_out), lambda i: (i, 0)),
            pl.BlockSpec((_BLK, 1), lambda i: (i, 0)),
        ],
        out_shape=[
            jax.ShapeDtypeStruct((N_PAD, d_out), jnp.float32),
            jax.ShapeDtypeStruct((N_PAD, 1), jnp.float32),
        ],
    )(p, hist, x, Wl, Wr, b)


def _tc_layerN(p, deg, h, Wl, Wr, b, act):
    d_out = Wl.shape[1]
    return pl.pallas_call(
        functools.partial(_layerN_body, act),
        grid=(N_PAD // _BLK,),
        in_specs=[
            pl.BlockSpec((NC, _BLK, D_FEAT), lambda i: (0, i, 0)),
            pl.BlockSpec((_BLK, 1), lambda i: (i, 0)),
            pl.BlockSpec((_BLK, D_FEAT), lambda i: (i, 0)),
            pl.BlockSpec(Wl.shape, lambda i: (0, 0)),
            pl.BlockSpec(Wr.shape, lambda i: (0, 0)),
            pl.BlockSpec(b.shape, lambda i: (0,)),
        ],
        out_specs=pl.BlockSpec((_BLK, d_out), lambda i: (i, 0)),
        out_shape=jax.ShapeDtypeStruct((N_PAD, d_out), jnp.float32),
    )(p, deg, h, Wl, Wr, b)


def kernel(x, edge_index, Wl1, Wr1, b1, Wl2, Wr2, b2, Wl3, Wr3, b3):
    ei = edge_index.astype(jnp.int32)
    src_t = jnp.pad(ei[0].reshape(NW, EDGES_PER_TILE), ((0, 0), (0, PAD_EDGES)))
    dst_t = jnp.pad(ei[1].reshape(NW, EDGES_PER_TILE), ((0, 0), (0, PAD_EDGES)),
                    constant_values=N_NODES)
    edges = (src_t * 16384 + dst_t).reshape(NW, N_CHUNKS, CHUNK)
    xp = jnp.pad(x, ((0, N_PAD - N_NODES), (0, 0)))
    z = jnp.zeros((N_PAD, D_FEAT), jnp.float32)

    hist = _sc_degree(edges)
    p1 = _sc_aggregate(xp, edges, z)
    h1, deg = _tc_layer1(p1, hist.T, xp, Wl1, Wr1, b1)
    p2 = _sc_aggregate(h1, edges, z)
    h2 = _tc_layerN(p2, deg, h1, Wl2, Wr2, b2, "relu")
    p3 = _sc_aggregate(h2, edges, z)
    return _tc_layerN(p3, deg, h2, Wl3, Wr3, b3, "sigmoid")[:N_NODES]


# speedup vs baseline: 1.9691x; 1.0010x over previous
import dataclasses
import functools

import jax
import jax.numpy as jnp
from jax import lax
from jax.experimental import pallas as pl
from jax.experimental.pallas import tpu as pltpu
from jax.experimental.pallas import tpu_sc as plsc

N_NODES = 10000
N_PAD = 10112
N_EDGES = 320000
D_FEAT = 128
NC = 2
NS = 16
NW = NC * NS
EDGES_PER_TILE = N_EDGES // NW
CHUNK = 80
N_CHUNKS = -(-EDGES_PER_TILE // CHUNK)

PAD_EDGES = N_CHUNKS * CHUNK - EDGES_PER_TILE
ROWS_PER_TILE = N_PAD // NS


def _unpack_idx(packed_v, j, i_v):
    for k in range(CHUNK // 16):
        p = packed_v[j, pl.ds(k * 16, 16)]
        i_v[0, pl.ds(k * 16, 16)] = lax.shift_right_logical(p, 14)
        i_v[1, pl.ds(k * 16, 16)] = lax.bitwise_and(p, 16383)


def _sc_aggregate(h, edges, zeros):
    mesh = plsc.VectorSubcoreMesh(core_axis_name="c", subcore_axis_name="s")

    @functools.partial(
        pl.kernel,
        mesh=mesh,
        out_type=jax.ShapeDtypeStruct((NC, N_PAD, D_FEAT), jnp.float32),
        scratch_types=[
            pltpu.VMEM((N_CHUNKS, CHUNK), jnp.int32),
            pltpu.VMEM((2, CHUNK), jnp.int32),
            pltpu.VMEM((2, CHUNK), jnp.int32),
            pltpu.VMEM((CHUNK, D_FEAT), jnp.float32),
            pltpu.VMEM((CHUNK, D_FEAT), jnp.float32),
            pltpu.VMEM_SHARED((N_PAD, D_FEAT), jnp.float32),
            pltpu.SemaphoreType.DMA,
            pltpu.SemaphoreType.DMA,
            pltpu.SemaphoreType.DMA,
            pltpu.SemaphoreType.DMA,
            pltpu.SemaphoreType.DMA,
        ],
    )
    def k(h_hbm, e_hbm, z_hbm, out_hbm, packed_v, i0, i1, rows0, rows1,
          acc_sh, g0, g1, s0, s1, zs):
        c = lax.axis_index("c")
        s = lax.axis_index("s")
        wid = c * NS + s
        row0 = s * ROWS_PER_TILE
        pltpu.async_copy(z_hbm.at[pl.ds(row0, ROWS_PER_TILE)],
                         acc_sh.at[pl.ds(row0, ROWS_PER_TILE)], zs)
        pltpu.sync_copy(e_hbm.at[wid], packed_v)
        _unpack_idx(packed_v, 0, i0)
        pltpu.async_copy(h_hbm.at[i0.at[0]], rows0, g0)
        _unpack_idx(packed_v, 1, i1)
        pltpu.async_copy(h_hbm.at[i1.at[0]], rows1, g1)
        pltpu.make_async_copy(z_hbm.at[pl.ds(row0, ROWS_PER_TILE)],
                              acc_sh.at[pl.ds(row0, ROWS_PER_TILE)], zs).wait()
        plsc.subcore_barrier()

        @pl.loop(0, N_CHUNKS, step=2)
        def _(j):
            @pl.when(j > 0)
            def _():
                pltpu.make_async_copy(rows0, acc_sh.at[i0.at[1]], s0).wait()
                _unpack_idx(packed_v, j, i0)
                pltpu.async_copy(h_hbm.at[i0.at[0]], rows0, g0)

            @pl.when(jnp.logical_and(j > 0, j + 1 < N_CHUNKS))
            def _():
                pltpu.make_async_copy(rows1, acc_sh.at[i1.at[1]], s1).wait()
                _unpack_idx(packed_v, j + 1, i1)
                pltpu.async_copy(h_hbm.at[i1.at[0]], rows1, g1)

            pltpu.make_async_copy(h_hbm.at[i0.at[0]], rows0, g0).wait()
            pltpu.async_copy(rows0, acc_sh.at[i0.at[1]], s0, add=True)

            @pl.when(j + 1 < N_CHUNKS)
            def _():
                pltpu.make_async_copy(h_hbm.at[i1.at[0]], rows1, g1).wait()
                pltpu.async_copy(rows1, acc_sh.at[i1.at[1]], s1, add=True)

        pltpu.make_async_copy(rows0, acc_sh.at[i0.at[1]], s0).wait()
        if N_CHUNKS > 1:
            pltpu.make_async_copy(rows1, acc_sh.at[i1.at[1]], s1).wait()
        plsc.subcore_barrier()
        pltpu.sync_copy(acc_sh.at[pl.ds(row0, ROWS_PER_TILE)],
                        out_hbm.at[c, pl.ds(row0, ROWS_PER_TILE)])

    return k(h, edges, zeros)


def _sc_degree(edges):
    mesh = plsc.VectorSubcoreMesh(core_axis_name="c", subcore_axis_name="s")
    cp = pltpu.CompilerParams()
    if "needs_layout_passes" in pltpu.CompilerParams.__dataclass_fields__:
        cp = dataclasses.replace(cp, needs_layout_passes=False)

    @functools.partial(
        pl.kernel,
        mesh=mesh,
        compiler_params=cp,
        out_type=jax.ShapeDtypeStruct((NW, N_PAD), jnp.float32),
        scratch_types=[
            pltpu.VMEM((N_CHUNKS * CHUNK,), jnp.int32),
            pltpu.VMEM((N_PAD,), jnp.float32),
        ],
    )
    def k(e_hbm, out_hbm, packed_v, hist_v):
        c = lax.axis_index("c")
        s = lax.axis_index("s")
        wid = c * NS + s
        pltpu.sync_copy(e_hbm.at[wid], packed_v)

        @pl.loop(0, N_PAD, step=16)
        def _(r):
            hist_v[pl.ds(r, 16)] = jnp.zeros((16,), jnp.float32)

        ones16 = jnp.ones((16,), jnp.float32)

        @pl.loop(0, N_CHUNKS * CHUNK, step=16)
        def _(e):
            p = packed_v[pl.ds(e, 16)]
            d = lax.bitwise_and(p, 16383)
            plsc.addupdate_scatter(hist_v, [d], ones16)

        pltpu.sync_copy(hist_v, out_hbm.at[wid])

    return k(edges.reshape(NW, N_CHUNKS * CHUNK))


def _layer1_body(p_ref, hist_ref, x_ref, wl_ref, wr_ref, b_ref, o_ref, deg_ref):
    deg = jnp.maximum(jnp.sum(hist_ref[...], axis=1), 1.0)[:, None]
    agg = (p_ref[0] + p_ref[1]) / deg
    out = (jnp.dot(agg, wl_ref[...], preferred_element_type=jnp.float32)
           + jnp.dot(x_ref[...], wr_ref[...], preferred_element_type=jnp.float32)
           + b_ref[...][None, :])
    o_ref[...] = jnp.maximum(out, 0.0)
    deg_ref[...] = deg


def _layerN_body(act, p_ref, deg_ref, h_ref, wl_ref, wr_ref, b_ref, o_ref):
    agg = (p_ref[0] + p_ref[1]) / deg_ref[...]
    out = (jnp.dot(agg, wl_ref[...], preferred_element_type=jnp.float32)
           + jnp.dot(h_ref[...], wr_ref[...], preferred_element_type=jnp.float32)
           + b_ref[...][None, :])
    if act == "relu":
        out = jnp.maximum(out, 0.0)
    else:
        out = jax.nn.sigmoid(out)
    o_ref[...] = out


_BLK = 1264


def _tc_layer1(p, hist, x, Wl, Wr, b):
    d_out = Wl.shape[1]
    return pl.pallas_call(
        _layer1_body,
        grid=(N_PAD // _BLK,),
        in_specs=[
            pl.BlockSpec((NC, _BLK, D_FEAT), lambda i: (0, i, 0)),
            pl.BlockSpec((_BLK, NW), lambda i: (i, 0)),
            pl.BlockSpec((_BLK, D_FEAT), lambda i: (i, 0)),
            pl.BlockSpec(Wl.shape, lambda i: (0, 0)),
            pl.BlockSpec(Wr.shape, lambda i: (0, 0)),
            pl.BlockSpec(b.shape, lambda i: (0,)),
        ],
        out_specs=[
            pl.BlockSpec((_BLK, d_out), lambda i: (i, 0)),
            pl.BlockSpec((_BLK, 1), lambda i: (i, 0)),
        ],
        out_shape=[
            jax.ShapeDtypeStruct((N_PAD, d_out), jnp.float32),
            jax.ShapeDtypeStruct((N_PAD, 1), jnp.float32),
        ],
    )(p, hist, x, Wl, Wr, b)


def _tc_layerN(p, deg, h, Wl, Wr, b, act):
    d_out = Wl.shape[1]
    return pl.pallas_call(
        functools.partial(_layerN_body, act),
        grid=(N_PAD // _BLK,),
        in_specs=[
            pl.BlockSpec((NC, _BLK, D_FEAT), lambda i: (0, i, 0)),
            pl.BlockSpec((_BLK, 1), lambda i: (i, 0)),
            pl.BlockSpec((_BLK, D_FEAT), lambda i: (i, 0)),
            pl.BlockSpec(Wl.shape, lambda i: (0, 0)),
            pl.BlockSpec(Wr.shape, lambda i: (0, 0)),
            pl.BlockSpec(b.shape, lambda i: (0,)),
        ],
        out_specs=pl.BlockSpec((_BLK, d_out), lambda i: (i, 0)),
        out_shape=jax.ShapeDtypeStruct((N_PAD, d_out), jnp.float32),
    )(p, deg, h, Wl, Wr, b)


def kernel(x, edge_index, Wl1, Wr1, b1, Wl2, Wr2, b2, Wl3, Wr3, b3):
    ei = edge_index.astype(jnp.int32)
    src_t = jnp.pad(ei[0].reshape(NW, EDGES_PER_TILE), ((0, 0), (0, PAD_EDGES)))
    dst_t = jnp.pad(ei[1].reshape(NW, EDGES_PER_TILE), ((0, 0), (0, PAD_EDGES)),
                    constant_values=N_NODES)
    edges = (src_t * 16384 + dst_t).reshape(NW, N_CHUNKS, CHUNK)
    xp = jnp.pad(x, ((0, N_PAD - N_NODES), (0, 0)))
    z = jnp.zeros((N_PAD, D_FEAT), jnp.float32)

    hist = _sc_degree(edges)
    p1 = _sc_aggregate(xp, edges, z)
    h1, deg = _tc_layer1(p1, hist.T, xp, Wl1, Wr1, b1)
    p2 = _sc_aggregate(h1, edges, z)
    h2 = _tc_layerN(p2, deg, h1, Wl2, Wr2, b2, "relu")
    p3 = _sc_aggregate(h2, edges, z)
    return _tc_layerN(p3, deg, h2, Wl3, Wr3, b3, "sigmoid")[:N_NODES]
